# baseline (speedup 1.0000x reference)
import functools

import jax
import jax.numpy as jnp
from jax import lax
from jax.experimental import pallas as pl
from jax.experimental.pallas import tpu as pltpu
from jax.experimental.pallas import tpu_sc as plsc

N = 10000
N_PAD = 10112
E = 320000
D_IN = 128
D_HID = 128
D_OUT = 64

NUM_CORES = 2
NUM_SUBCORES = 16
NUM_WORKERS = NUM_CORES * NUM_SUBCORES
EDGE_B = 256
ROWS_C0 = 72
ROWS_C1 = 8
T_ROWS = NUM_SUBCORES * (ROWS_C0 + ROWS_C1)
E_PAD = T_ROWS * EDGE_B
D_CNT = 32
SUB_ROWS = N_PAD // NUM_SUBCORES

ROW_BLK = 1264


@functools.lru_cache(maxsize=None)
def _make_segsum(d, with_count):
    mesh = plsc.VectorSubcoreMesh(core_axis_name="c", subcore_axis_name="s")

    out_type = [jax.ShapeDtypeStruct((NUM_CORES, N_PAD, d), jnp.bfloat16)]
    scratch = [
        pltpu.VMEM((ROWS_C0, EDGE_B), jnp.int32),
        pltpu.VMEM((ROWS_C0, EDGE_B), jnp.int32),
        pltpu.VMEM((EDGE_B, d), jnp.bfloat16),
        pltpu.VMEM((EDGE_B, d), jnp.bfloat16),
        pltpu.VMEM_SHARED((N_PAD, d), jnp.bfloat16),
        pltpu.SemaphoreType.DMA,
        pltpu.SemaphoreType.DMA,
    ]
    if with_count:
        out_type.append(
            jax.ShapeDtypeStruct((NUM_CORES, N_PAD, D_CNT), jnp.bfloat16))
        scratch += [
            pltpu.VMEM((EDGE_B, D_CNT), jnp.bfloat16),
            pltpu.VMEM_SHARED((N_PAD, D_CNT), jnp.bfloat16),
        ]

    @functools.partial(
        pl.kernel,
        mesh=mesh,
        compiler_params=pltpu.CompilerParams(use_tc_tiling_on_sc=False),
        out_type=out_type,
        scratch_types=scratch,
    )
    def segsum(feat, src2, dst2, zeros, *rest):
        if with_count:
            (zc, ones, out, outc, sidx, didx, rows0, rows1, agg, sem0, sem1,
             onesv, cnt) = rest
        else:
            out, sidx, didx, rows0, rows1, agg, sem0, sem1 = rest
        c = lax.axis_index("c")
        s = lax.axis_index("s")

        def run_edges(base, nrows):
            pltpu.sync_copy(src2.at[pl.ds(base, nrows)], sidx.at[pl.ds(0, nrows)])
            pltpu.sync_copy(dst2.at[pl.ds(base, nrows)], didx.at[pl.ds(0, nrows)])
            pltpu.async_copy(feat.at[sidx.at[0]], rows0, sem0)
            pltpu.async_copy(feat.at[sidx.at[1]], rows1, sem1)

            def body(i, carry):
                t0 = 2 * i
                pltpu.make_async_copy(feat.at[sidx.at[t0]], rows0, sem0).wait()
                pltpu.sync_copy(rows0, agg.at[didx.at[t0]], add=True)
                if with_count:
                    pltpu.sync_copy(onesv, cnt.at[didx.at[t0]], add=True)

                @pl.when(i < nrows // 2 - 1)
                def _():
                    pltpu.async_copy(feat.at[sidx.at[t0 + 2]], rows0, sem0)

                pltpu.make_async_copy(feat.at[sidx.at[t0 + 1]], rows1, sem1).wait()
                pltpu.sync_copy(rows1, agg.at[didx.at[t0 + 1]], add=True)
                if with_count:
                    pltpu.sync_copy(onesv, cnt.at[didx.at[t0 + 1]], add=True)

                @pl.when(i < nrows // 2 - 1)
                def _():
                    pltpu.async_copy(feat.at[sidx.at[t0 + 3]], rows1, sem1)

                return carry

            lax.fori_loop(0, nrows // 2, body, 0)

        def zero_stripe(buf, seed_rows):
            base = s * SUB_ROWS
            pltpu.sync_copy(seed_rows, buf.at[pl.ds(base, 8)])
            done = 8
            while done < SUB_ROWS:
                step = min(done, SUB_ROWS - done)
                pltpu.sync_copy(buf.at[pl.ds(base, step)],
                                buf.at[pl.ds(base + done, step)])
                done += step

        zero_stripe(agg, zeros)
        if with_count:
            zero_stripe(cnt, zc)
            pltpu.sync_copy(ones, onesv)
        plsc.subcore_barrier()

        @pl.when(c == 0)
        def _():
            run_edges(s * ROWS_C0, ROWS_C0)

        @pl.when(c == 1)
        def _():
            run_edges(NUM_SUBCORES * ROWS_C0 + s * ROWS_C1, ROWS_C1)

        plsc.subcore_barrier()
        pltpu.sync_copy(
            agg.at[pl.ds(s * SUB_ROWS, SUB_ROWS)],
            out.at[c, pl.ds(s * SUB_ROWS, SUB_ROWS)],
        )
        if with_count:
            pltpu.sync_copy(
                cnt.at[pl.ds(s * SUB_ROWS, SUB_ROWS)],
                outc.at[c, pl.ds(s * SUB_ROWS, SUB_ROWS)],
            )

    return segsum


def _segsum_l1(*args):
    return _make_segsum(D_HID, True)(*args)


def _segsum_l2(*args):
    return _make_segsum(D_OUT, False)(*args)


def _tc1_body(x_ref, w_ref, y1_ref, y2_ref):
    y = jnp.dot(x_ref[...], w_ref[...], preferred_element_type=jnp.float32)
    y1_ref[...] = y[:, :D_HID].astype(jnp.bfloat16)
    y2_ref[...] = y[:, D_HID:]


def _tc1(xp, wcat):
    grid = (N_PAD // ROW_BLK,)
    return pl.pallas_call(
        _tc1_body,
        grid=grid,
        in_specs=[
            pl.BlockSpec((ROW_BLK, D_IN), lambda i: (i, 0)),
            pl.BlockSpec((D_IN, 2 * D_HID), lambda i: (0, 0)),
        ],
        out_specs=[
            pl.BlockSpec((ROW_BLK, D_HID), lambda i: (i, 0)),
            pl.BlockSpec((ROW_BLK, D_HID), lambda i: (i, 0)),
        ],
        out_shape=[
            jax.ShapeDtypeStruct((N_PAD, D_HID), jnp.bfloat16),
            jax.ShapeDtypeStruct((N_PAD, D_HID), jnp.float32),
        ],
    )(xp, wcat)


def _tc2_body(p0_ref, p1_ref, c0_ref, c1_ref, y2_ref, b1_ref, w2_ref,
              g_ref, hr_ref, rinv_ref):
    a = p0_ref[...].astype(jnp.float32) + p1_ref[...].astype(jnp.float32)
    cnt = c0_ref[...].astype(jnp.float32) + c1_ref[...].astype(jnp.float32)
    inv = 1.0 / jnp.maximum(cnt[:, :1], 1.0)
    h = jnp.maximum(a * inv + b1_ref[...] + y2_ref[...], 0.0)
    hw = jnp.dot(h, w2_ref[...], preferred_element_type=jnp.float32)
    g_ref[...] = hw[:, :D_OUT].astype(jnp.bfloat16)
    hr_ref[...] = hw[:, D_OUT:]
    rinv_ref[...] = jnp.broadcast_to(inv, (a.shape[0], D_OUT))


def _tc2(p0, p1, c0, c1, y2, b1, w2cat):
    grid = (N_PAD // ROW_BLK,)
    return pl.pallas_call(
        _tc2_body,
        grid=grid,
        in_specs=[
            pl.BlockSpec((ROW_BLK, D_HID), lambda i: (i, 0)),
            pl.BlockSpec((ROW_BLK, D_HID), lambda i: (i, 0)),
            pl.BlockSpec((ROW_BLK, D_CNT), lambda i: (i, 0)),
            pl.BlockSpec((ROW_BLK, D_CNT), lambda i: (i, 0)),
            pl.BlockSpec((ROW_BLK, D_HID), lambda i: (i, 0)),
            pl.BlockSpec((1, D_HID), lambda i: (0, 0)),
            pl.BlockSpec((D_HID, 2 * D_OUT), lambda i: (0, 0)),
        ],
        out_specs=[
            pl.BlockSpec((ROW_BLK, D_OUT), lambda i: (i, 0)),
            pl.BlockSpec((ROW_BLK, D_OUT), lambda i: (i, 0)),
            pl.BlockSpec((ROW_BLK, D_OUT), lambda i: (i, 0)),
        ],
        out_shape=[
            jax.ShapeDtypeStruct((N_PAD, D_OUT), jnp.bfloat16),
            jax.ShapeDtypeStruct((N_PAD, D_OUT), jnp.float32),
            jax.ShapeDtypeStruct((N_PAD, D_OUT), jnp.float32),
        ],
    )(p0, p1, c0, c1, y2, b1, w2cat)


def _tc3_body(q0_ref, q1_ref, rinv_ref, hr_ref, b2_ref, z_ref):
    z_ref[...] = (
        (q0_ref[...].astype(jnp.float32) + q1_ref[...].astype(jnp.float32))
        * rinv_ref[...] + b2_ref[...] + hr_ref[...]
    )


def _tc3(q0, q1, rinv, hr, b2):
    grid = (N_PAD // ROW_BLK,)
    spec = pl.BlockSpec((ROW_BLK, D_OUT), lambda i: (i, 0))
    return pl.pallas_call(
        _tc3_body,
        grid=grid,
        in_specs=[spec, spec, spec, spec, pl.BlockSpec((1, D_OUT), lambda i: (0, 0))],
        out_specs=spec,
        out_shape=jax.ShapeDtypeStruct((N_PAD, D_OUT), jnp.float32),
    )(q0, q1, rinv, hr, b2)


def kernel(x, edge_index, W1l, b1, W1r, W2l, b2, W2r):
    src = edge_index[0].astype(jnp.int32)
    dst = edge_index[1].astype(jnp.int32)
    pad = E_PAD - E
    src2 = jnp.concatenate([src, jnp.zeros((pad,), jnp.int32)]).reshape(T_ROWS, EDGE_B)
    dst2 = jnp.concatenate([dst, jnp.full((pad,), N, jnp.int32)]).reshape(T_ROWS, EDGE_B)
    xp = jnp.pad(x, ((0, N_PAD - N), (0, 0)))

    y1a, y2 = _tc1(xp, jnp.concatenate([W1l, W1r], axis=1))
    agg1, cnt1 = _segsum_l1(
        y1a, src2, dst2,
        jnp.zeros((8, D_HID), jnp.bfloat16),
        jnp.zeros((8, D_CNT), jnp.bfloat16),
        jnp.ones((EDGE_B, D_CNT), jnp.bfloat16),
    )
    g, hr, rinv = _tc2(agg1[0], agg1[1], cnt1[0], cnt1[1], y2,
                       b1.reshape(1, D_HID),
                       jnp.concatenate([W2l, W2r], axis=1))
    (agg2,) = _segsum_l2(g, src2, dst2, jnp.zeros((8, D_OUT), jnp.bfloat16))
    z = _tc3(agg2[0], agg2[1], rinv, hr, b2.reshape(1, D_OUT))
    return z[:N]


# speedup vs baseline: 1.5685x; 1.5685x over previous
import functools

import jax
import jax.numpy as jnp
from jax import lax
from jax.experimental import pallas as pl
from jax.experimental.pallas import tpu as pltpu
from jax.experimental.pallas import tpu_sc as plsc

N = 10000
N_PAD = 10112
E = 320000
D_IN = 128
D_HID = 128
D_OUT = 64

NUM_CORES = 2
NUM_SUBCORES = 16
NUM_WORKERS = NUM_CORES * NUM_SUBCORES
EDGE_B = 256
ROWS_C0 = 72
ROWS_C1 = 8
T_ROWS = NUM_SUBCORES * (ROWS_C0 + ROWS_C1)
E_PAD = T_ROWS * EDGE_B
D_CNT = 32
SUB_ROWS = N_PAD // NUM_SUBCORES

ROW_BLK = 1264


@functools.lru_cache(maxsize=None)
def _make_segsum(d, with_count):
    mesh = plsc.VectorSubcoreMesh(core_axis_name="c", subcore_axis_name="s")

    out_type = [jax.ShapeDtypeStruct((NUM_CORES, N_PAD, d), jnp.bfloat16)]
    scratch = [
        pltpu.VMEM((ROWS_C0, EDGE_B), jnp.int32),
        pltpu.VMEM((ROWS_C0, EDGE_B), jnp.int32),
        pltpu.VMEM((EDGE_B, d), jnp.bfloat16),
        pltpu.VMEM((EDGE_B, d), jnp.bfloat16),
        pltpu.VMEM_SHARED((N_PAD, d), jnp.bfloat16),
        pltpu.SemaphoreType.DMA,
        pltpu.SemaphoreType.DMA,
    ]
    if with_count:
        out_type.append(
            jax.ShapeDtypeStruct((NUM_CORES, N_PAD, D_CNT), jnp.bfloat16))
        scratch += [
            pltpu.VMEM((EDGE_B, D_CNT), jnp.bfloat16),
            pltpu.VMEM_SHARED((N_PAD, D_CNT), jnp.bfloat16),
        ]

    @functools.partial(
        pl.kernel,
        mesh=mesh,
        compiler_params=pltpu.CompilerParams(use_tc_tiling_on_sc=False),
        out_type=out_type,
        scratch_types=scratch,
    )
    def segsum(feat, src2, dst2, zeros, *rest):
        if with_count:
            (zc, ones, out, outc, sidx, didx, rows0, rows1, agg, sem0, sem1,
             onesv, cnt) = rest
        else:
            out, sidx, didx, rows0, rows1, agg, sem0, sem1 = rest
        c = lax.axis_index("c")
        s = lax.axis_index("s")

        def run_edges(base, nrows):
            pltpu.sync_copy(src2.at[pl.ds(base, nrows)], sidx.at[pl.ds(0, nrows)])
            pltpu.sync_copy(dst2.at[pl.ds(base, nrows)], didx.at[pl.ds(0, nrows)])
            pltpu.async_copy(feat.at[sidx.at[0]], rows0, sem0)
            pltpu.async_copy(feat.at[sidx.at[1]], rows1, sem1)

            def body(i, carry):
                t0 = 2 * i
                pltpu.make_async_copy(feat.at[sidx.at[t0]], rows0, sem0).wait()
                pltpu.sync_copy(rows0, agg.at[didx.at[t0]], add=True)
                if with_count:
                    pltpu.sync_copy(onesv, cnt.at[didx.at[t0]], add=True)

                @pl.when(i < nrows // 2 - 1)
                def _():
                    pltpu.async_copy(feat.at[sidx.at[t0 + 2]], rows0, sem0)

                pltpu.make_async_copy(feat.at[sidx.at[t0 + 1]], rows1, sem1).wait()
                pltpu.sync_copy(rows1, agg.at[didx.at[t0 + 1]], add=True)
                if with_count:
                    pltpu.sync_copy(onesv, cnt.at[didx.at[t0 + 1]], add=True)

                @pl.when(i < nrows // 2 - 1)
                def _():
                    pltpu.async_copy(feat.at[sidx.at[t0 + 3]], rows1, sem1)

                return carry

            lax.fori_loop(0, nrows // 2, body, 0)

        pltpu.sync_copy(zeros, agg.at[pl.ds(s * SUB_ROWS, SUB_ROWS)])
        if with_count:
            pltpu.sync_copy(zc, cnt.at[pl.ds(s * SUB_ROWS, SUB_ROWS)])
            pltpu.sync_copy(ones, onesv)
        plsc.subcore_barrier()

        @pl.when(c == 0)
        def _():
            run_edges(s * ROWS_C0, ROWS_C0)

        @pl.when(c == 1)
        def _():
            run_edges(NUM_SUBCORES * ROWS_C0 + s * ROWS_C1, ROWS_C1)

        plsc.subcore_barrier()
        pltpu.sync_copy(
            agg.at[pl.ds(s * SUB_ROWS, SUB_ROWS)],
            out.at[c, pl.ds(s * SUB_ROWS, SUB_ROWS)],
        )
        if with_count:
            pltpu.sync_copy(
                cnt.at[pl.ds(s * SUB_ROWS, SUB_ROWS)],
                outc.at[c, pl.ds(s * SUB_ROWS, SUB_ROWS)],
            )

    return segsum


def _segsum_l1(*args):
    return _make_segsum(D_HID, True)(*args)


def _segsum_l2(*args):
    return _make_segsum(D_OUT, False)(*args)


def _tc1_body(x_ref, w_ref, y1_ref, y2_ref):
    y = jnp.dot(x_ref[...], w_ref[...], preferred_element_type=jnp.float32)
    y1_ref[...] = y[:, :D_HID].astype(jnp.bfloat16)
    y2_ref[...] = y[:, D_HID:]


def _tc1(xp, wcat):
    grid = (N_PAD // ROW_BLK,)
    return pl.pallas_call(
        _tc1_body,
        grid=grid,
        in_specs=[
            pl.BlockSpec((ROW_BLK, D_IN), lambda i: (i, 0)),
            pl.BlockSpec((D_IN, 2 * D_HID), lambda i: (0, 0)),
        ],
        out_specs=[
            pl.BlockSpec((ROW_BLK, D_HID), lambda i: (i, 0)),
            pl.BlockSpec((ROW_BLK, D_HID), lambda i: (i, 0)),
        ],
        out_shape=[
            jax.ShapeDtypeStruct((N_PAD, D_HID), jnp.bfloat16),
            jax.ShapeDtypeStruct((N_PAD, D_HID), jnp.float32),
        ],
    )(xp, wcat)


def _tc2_body(p0_ref, p1_ref, c0_ref, c1_ref, y2_ref, b1_ref, w2_ref,
              g_ref, hr_ref, rinv_ref):
    a = p0_ref[...].astype(jnp.float32) + p1_ref[...].astype(jnp.float32)
    cnt = c0_ref[...].astype(jnp.float32) + c1_ref[...].astype(jnp.float32)
    inv = 1.0 / jnp.maximum(cnt[:, :1], 1.0)
    h = jnp.maximum(a * inv + b1_ref[...] + y2_ref[...], 0.0)
    hw = jnp.dot(h, w2_ref[...], preferred_element_type=jnp.float32)
    g_ref[...] = hw[:, :D_OUT].astype(jnp.bfloat16)
    hr_ref[...] = hw[:, D_OUT:]
    rinv_ref[...] = jnp.broadcast_to(inv, (a.shape[0], D_OUT))


def _tc2(p0, p1, c0, c1, y2, b1, w2cat):
    grid = (N_PAD // ROW_BLK,)
    return pl.pallas_call(
        _tc2_body,
        grid=grid,
        in_specs=[
            pl.BlockSpec((ROW_BLK, D_HID), lambda i: (i, 0)),
            pl.BlockSpec((ROW_BLK, D_HID), lambda i: (i, 0)),
            pl.BlockSpec((ROW_BLK, D_CNT), lambda i: (i, 0)),
            pl.BlockSpec((ROW_BLK, D_CNT), lambda i: (i, 0)),
            pl.BlockSpec((ROW_BLK, D_HID), lambda i: (i, 0)),
            pl.BlockSpec((1, D_HID), lambda i: (0, 0)),
            pl.BlockSpec((D_HID, 2 * D_OUT), lambda i: (0, 0)),
        ],
        out_specs=[
            pl.BlockSpec((ROW_BLK, D_OUT), lambda i: (i, 0)),
            pl.BlockSpec((ROW_BLK, D_OUT), lambda i: (i, 0)),
            pl.BlockSpec((ROW_BLK, D_OUT), lambda i: (i, 0)),
        ],
        out_shape=[
            jax.ShapeDtypeStruct((N_PAD, D_OUT), jnp.bfloat16),
            jax.ShapeDtypeStruct((N_PAD, D_OUT), jnp.float32),
            jax.ShapeDtypeStruct((N_PAD, D_OUT), jnp.float32),
        ],
    )(p0, p1, c0, c1, y2, b1, w2cat)


def _tc3_body(q0_ref, q1_ref, rinv_ref, hr_ref, b2_ref, z_ref):
    z_ref[...] = (
        (q0_ref[...].astype(jnp.float32) + q1_ref[...].astype(jnp.float32))
        * rinv_ref[...] + b2_ref[...] + hr_ref[...]
    )


def _tc3(q0, q1, rinv, hr, b2):
    grid = (N_PAD // ROW_BLK,)
    spec = pl.BlockSpec((ROW_BLK, D_OUT), lambda i: (i, 0))
    return pl.pallas_call(
        _tc3_body,
        grid=grid,
        in_specs=[spec, spec, spec, spec, pl.BlockSpec((1, D_OUT), lambda i: (0, 0))],
        out_specs=spec,
        out_shape=jax.ShapeDtypeStruct((N_PAD, D_OUT), jnp.float32),
    )(q0, q1, rinv, hr, b2)


def kernel(x, edge_index, W1l, b1, W1r, W2l, b2, W2r):
    src = edge_index[0].astype(jnp.int32)
    dst = edge_index[1].astype(jnp.int32)
    pad = E_PAD - E
    src2 = jnp.concatenate([src, jnp.zeros((pad,), jnp.int32)]).reshape(T_ROWS, EDGE_B)
    dst2 = jnp.concatenate([dst, jnp.full((pad,), N, jnp.int32)]).reshape(T_ROWS, EDGE_B)
    xp = jnp.pad(x, ((0, N_PAD - N), (0, 0)))

    y1a, y2 = _tc1(xp, jnp.concatenate([W1l, W1r], axis=1))
    agg1, cnt1 = _segsum_l1(
        y1a, src2, dst2,
        jnp.zeros((SUB_ROWS, D_HID), jnp.bfloat16),
        jnp.zeros((SUB_ROWS, D_CNT), jnp.bfloat16),
        jnp.ones((EDGE_B, D_CNT), jnp.bfloat16),
    )
    g, hr, rinv = _tc2(agg1[0], agg1[1], cnt1[0], cnt1[1], y2,
                       b1.reshape(1, D_HID),
                       jnp.concatenate([W2l, W2r], axis=1))
    (agg2,) = _segsum_l2(g, src2, dst2, jnp.zeros((SUB_ROWS, D_OUT), jnp.bfloat16))
    z = _tc3(agg2[0], agg2[1], rinv, hr, b2.reshape(1, D_OUT))
    return z[:N]


# speedup vs baseline: 1.5790x; 1.0067x over previous
import functools

import jax
import jax.numpy as jnp
from jax import lax
from jax.experimental import pallas as pl
from jax.experimental.pallas import tpu as pltpu
from jax.experimental.pallas import tpu_sc as plsc

N = 10000
N_PAD = 10112
E = 320000
D_IN = 128
D_HID = 128
D_OUT = 64

NUM_CORES = 2
NUM_SUBCORES = 16
NUM_WORKERS = NUM_CORES * NUM_SUBCORES
EDGE_B = 256
ROWS_C0 = 72
ROWS_C1 = 8
T_ROWS = NUM_SUBCORES * (ROWS_C0 + ROWS_C1)
E_PAD = T_ROWS * EDGE_B
D_CNT = 32
SUB_ROWS = N_PAD // NUM_SUBCORES

ROW_BLK = 1264


@functools.lru_cache(maxsize=None)
def _make_segsum(d, with_count):
    mesh = plsc.VectorSubcoreMesh(core_axis_name="c", subcore_axis_name="s")

    out_type = [jax.ShapeDtypeStruct((NUM_CORES, N_PAD, d), jnp.bfloat16)]
    scratch = [
        pltpu.VMEM((ROWS_C0, EDGE_B), jnp.int32),
        pltpu.VMEM((ROWS_C0, EDGE_B), jnp.int32),
        pltpu.VMEM((EDGE_B, d), jnp.bfloat16),
        pltpu.VMEM((EDGE_B, d), jnp.bfloat16),
        pltpu.VMEM_SHARED((N_PAD, d), jnp.bfloat16),
        pltpu.SemaphoreType.DMA,
        pltpu.SemaphoreType.DMA,
        pltpu.SemaphoreType.DMA,
        pltpu.SemaphoreType.DMA,
        pltpu.SemaphoreType.DMA,
        pltpu.SemaphoreType.DMA,
    ]
    if with_count:
        out_type.append(
            jax.ShapeDtypeStruct((NUM_CORES, N_PAD, D_CNT), jnp.bfloat16))
        scratch += [
            pltpu.VMEM((EDGE_B, D_CNT), jnp.bfloat16),
            pltpu.VMEM_SHARED((N_PAD, D_CNT), jnp.bfloat16),
        ]

    @functools.partial(
        pl.kernel,
        mesh=mesh,
        compiler_params=pltpu.CompilerParams(use_tc_tiling_on_sc=False),
        out_type=out_type,
        scratch_types=scratch,
    )
    def segsum(feat, src2, dst2, zeros, *rest):
        if with_count:
            (zc, ones, out, outc, sidx, didx, rows0, rows1, agg, sem0, sem1,
             sem2, sem3, sem4, sem5, onesv, cnt) = rest
        else:
            (out, sidx, didx, rows0, rows1, agg, sem0, sem1, sem2, sem3,
             sem4, sem5) = rest
        sems = (sem0, sem1, sem2, sem3)
        chunks = ((0, 160), (160, 160), (320, 160), (480, 152))
        c = lax.axis_index("c")
        s = lax.axis_index("s")

        def run_edges(base, nrows):
            pltpu.sync_copy(src2.at[pl.ds(base, nrows)], sidx.at[pl.ds(0, nrows)])
            pltpu.sync_copy(dst2.at[pl.ds(base, nrows)], didx.at[pl.ds(0, nrows)])
            pltpu.async_copy(feat.at[sidx.at[0]], rows0, sem0)
            pltpu.async_copy(feat.at[sidx.at[1]], rows1, sem1)

            def body(i, carry):
                t0 = 2 * i
                pltpu.make_async_copy(feat.at[sidx.at[t0]], rows0, sem0).wait()
                pltpu.sync_copy(rows0, agg.at[didx.at[t0]], add=True)
                if with_count:
                    pltpu.sync_copy(onesv, cnt.at[didx.at[t0]], add=True)

                @pl.when(i < nrows // 2 - 1)
                def _():
                    pltpu.async_copy(feat.at[sidx.at[t0 + 2]], rows0, sem0)

                pltpu.make_async_copy(feat.at[sidx.at[t0 + 1]], rows1, sem1).wait()
                pltpu.sync_copy(rows1, agg.at[didx.at[t0 + 1]], add=True)
                if with_count:
                    pltpu.sync_copy(onesv, cnt.at[didx.at[t0 + 1]], add=True)

                @pl.when(i < nrows // 2 - 1)
                def _():
                    pltpu.async_copy(feat.at[sidx.at[t0 + 3]], rows1, sem1)

                return carry

            lax.fori_loop(0, nrows // 2, body, 0)

        base = s * SUB_ROWS
        for (off, sz), sem in zip(chunks, sems):
            pltpu.async_copy(zeros.at[pl.ds(off, sz)],
                             agg.at[pl.ds(base + off, sz)], sem)
        if with_count:
            pltpu.async_copy(zc, cnt.at[pl.ds(base, SUB_ROWS)], sem4)
            pltpu.async_copy(ones, onesv, sem5)
        for (off, sz), sem in zip(chunks, sems):
            pltpu.make_async_copy(zeros.at[pl.ds(off, sz)],
                                  agg.at[pl.ds(base + off, sz)], sem).wait()
        if with_count:
            pltpu.make_async_copy(zc, cnt.at[pl.ds(base, SUB_ROWS)], sem4).wait()
            pltpu.make_async_copy(ones, onesv, sem5).wait()
        plsc.subcore_barrier()

        @pl.when(c == 0)
        def _():
            run_edges(s * ROWS_C0, ROWS_C0)

        @pl.when(c == 1)
        def _():
            run_edges(NUM_SUBCORES * ROWS_C0 + s * ROWS_C1, ROWS_C1)

        plsc.subcore_barrier()
        for (off, sz), sem in zip(chunks, sems):
            pltpu.async_copy(agg.at[pl.ds(base + off, sz)],
                             out.at[c, pl.ds(base + off, sz)], sem)
        if with_count:
            pltpu.async_copy(cnt.at[pl.ds(base, SUB_ROWS)],
                             outc.at[c, pl.ds(base, SUB_ROWS)], sem4)
        for (off, sz), sem in zip(chunks, sems):
            pltpu.make_async_copy(agg.at[pl.ds(base + off, sz)],
                                  out.at[c, pl.ds(base + off, sz)], sem).wait()
        if with_count:
            pltpu.make_async_copy(cnt.at[pl.ds(base, SUB_ROWS)],
                                  outc.at[c, pl.ds(base, SUB_ROWS)], sem4).wait()

    return segsum


def _segsum_l1(*args):
    return _make_segsum(D_HID, True)(*args)


def _segsum_l2(*args):
    return _make_segsum(D_OUT, False)(*args)


def _tc1_body(x_ref, w_ref, y1_ref, y2_ref):
    y = jnp.dot(x_ref[...], w_ref[...], preferred_element_type=jnp.float32)
    y1_ref[...] = y[:, :D_HID].astype(jnp.bfloat16)
    y2_ref[...] = y[:, D_HID:]


def _tc1(xp, wcat):
    grid = (N_PAD // ROW_BLK,)
    return pl.pallas_call(
        _tc1_body,
        grid=grid,
        in_specs=[
            pl.BlockSpec((ROW_BLK, D_IN), lambda i: (i, 0)),
            pl.BlockSpec((D_IN, 2 * D_HID), lambda i: (0, 0)),
        ],
        out_specs=[
            pl.BlockSpec((ROW_BLK, D_HID), lambda i: (i, 0)),
            pl.BlockSpec((ROW_BLK, D_HID), lambda i: (i, 0)),
        ],
        out_shape=[
            jax.ShapeDtypeStruct((N_PAD, D_HID), jnp.bfloat16),
            jax.ShapeDtypeStruct((N_PAD, D_HID), jnp.float32),
        ],
    )(xp, wcat)


def _tc2_body(p0_ref, p1_ref, c0_ref, c1_ref, y2_ref, b1_ref, w2_ref,
              g_ref, hr_ref, rinv_ref):
    a = p0_ref[...].astype(jnp.float32) + p1_ref[...].astype(jnp.float32)
    cnt = c0_ref[...].astype(jnp.float32) + c1_ref[...].astype(jnp.float32)
    inv = 1.0 / jnp.maximum(cnt[:, :1], 1.0)
    h = jnp.maximum(a * inv + b1_ref[...] + y2_ref[...], 0.0)
    hw = jnp.dot(h, w2_ref[...], preferred_element_type=jnp.float32)
    g_ref[...] = hw[:, :D_OUT].astype(jnp.bfloat16)
    hr_ref[...] = hw[:, D_OUT:]
    rinv_ref[...] = jnp.broadcast_to(inv, (a.shape[0], D_OUT))


def _tc2(p0, p1, c0, c1, y2, b1, w2cat):
    grid = (N_PAD // ROW_BLK,)
    return pl.pallas_call(
        _tc2_body,
        grid=grid,
        in_specs=[
            pl.BlockSpec((ROW_BLK, D_HID), lambda i: (i, 0)),
            pl.BlockSpec((ROW_BLK, D_HID), lambda i: (i, 0)),
            pl.BlockSpec((ROW_BLK, D_CNT), lambda i: (i, 0)),
            pl.BlockSpec((ROW_BLK, D_CNT), lambda i: (i, 0)),
            pl.BlockSpec((ROW_BLK, D_HID), lambda i: (i, 0)),
            pl.BlockSpec((1, D_HID), lambda i: (0, 0)),
            pl.BlockSpec((D_HID, 2 * D_OUT), lambda i: (0, 0)),
        ],
        out_specs=[
            pl.BlockSpec((ROW_BLK, D_OUT), lambda i: (i, 0)),
            pl.BlockSpec((ROW_BLK, D_OUT), lambda i: (i, 0)),
            pl.BlockSpec((ROW_BLK, D_OUT), lambda i: (i, 0)),
        ],
        out_shape=[
            jax.ShapeDtypeStruct((N_PAD, D_OUT), jnp.bfloat16),
            jax.ShapeDtypeStruct((N_PAD, D_OUT), jnp.float32),
            jax.ShapeDtypeStruct((N_PAD, D_OUT), jnp.float32),
        ],
    )(p0, p1, c0, c1, y2, b1, w2cat)


def _tc3_body(q0_ref, q1_ref, rinv_ref, hr_ref, b2_ref, z_ref):
    z_ref[...] = (
        (q0_ref[...].astype(jnp.float32) + q1_ref[...].astype(jnp.float32))
        * rinv_ref[...] + b2_ref[...] + hr_ref[...]
    )


def _tc3(q0, q1, rinv, hr, b2):
    grid = (N_PAD // ROW_BLK,)
    spec = pl.BlockSpec((ROW_BLK, D_OUT), lambda i: (i, 0))
    return pl.pallas_call(
        _tc3_body,
        grid=grid,
        in_specs=[spec, spec, spec, spec, pl.BlockSpec((1, D_OUT), lambda i: (0, 0))],
        out_specs=spec,
        out_shape=jax.ShapeDtypeStruct((N_PAD, D_OUT), jnp.float32),
    )(q0, q1, rinv, hr, b2)


def kernel(x, edge_index, W1l, b1, W1r, W2l, b2, W2r):
    src = edge_index[0].astype(jnp.int32)
    dst = edge_index[1].astype(jnp.int32)
    pad = E_PAD - E
    src2 = jnp.concatenate([src, jnp.zeros((pad,), jnp.int32)]).reshape(T_ROWS, EDGE_B)
    dst2 = jnp.concatenate([dst, jnp.full((pad,), N, jnp.int32)]).reshape(T_ROWS, EDGE_B)
    xp = jnp.pad(x, ((0, N_PAD - N), (0, 0)))

    y1a, y2 = _tc1(xp, jnp.concatenate([W1l, W1r], axis=1))
    agg1, cnt1 = _segsum_l1(
        y1a, src2, dst2,
        jnp.zeros((SUB_ROWS, D_HID), jnp.bfloat16),
        jnp.zeros((SUB_ROWS, D_CNT), jnp.bfloat16),
        jnp.ones((EDGE_B, D_CNT), jnp.bfloat16),
    )
    g, hr, rinv = _tc2(agg1[0], agg1[1], cnt1[0], cnt1[1], y2,
                       b1.reshape(1, D_HID),
                       jnp.concatenate([W2l, W2r], axis=1))
    (agg2,) = _segsum_l2(g, src2, dst2, jnp.zeros((SUB_ROWS, D_OUT), jnp.bfloat16))
    z = _tc3(agg2[0], agg2[1], rinv, hr, b2.reshape(1, D_OUT))
    return z[:N]
